# baseline (speedup 1.0000x reference)
import jax
import jax.numpy as jnp
from jax import lax
from jax.experimental import pallas as pl
from jax.experimental.pallas import tpu as pltpu
from jax.experimental.pallas import tpu_sc as plsc

BATCH = 16384
EMBED = 128
LANES = 16

NUM_CORES = 2
NUM_SUBCORES = 16
NUM_WORKERS = NUM_CORES * NUM_SUBCORES
B_PER_W = BATCH // NUM_WORKERS
CHUNK = 32
N_CHUNKS = B_PER_W // CHUNK
NBUF = 8


def _body(uid_hbm, iid_hbm, p_hbm, q_hbm, y_hbm,
          uidx_v, iidx_v, p_rows, q_rows, out_v, sem_idx, sem_p, sem_q):
  wid = lax.axis_index("s") * NUM_CORES + lax.axis_index("c")
  base = wid * B_PER_W

  idx_copies = []
  for k in range(N_CHUNKS):
    idx_copies.append(pltpu.async_copy(
        uid_hbm.at[pl.ds(base + k * CHUNK, CHUNK)], uidx_v.at[k], sem_idx))
    idx_copies.append(pltpu.async_copy(
        iid_hbm.at[pl.ds(base + k * CHUNK, CHUNK)], iidx_v.at[k], sem_idx))
  for c in idx_copies:
    c.wait()

  def start_gather(k):
    buf = k % NBUF
    cp = pltpu.async_copy(p_hbm.at[uidx_v.at[k]], p_rows.at[buf], sem_p.at[buf])
    cq = pltpu.async_copy(q_hbm.at[iidx_v.at[k]], q_rows.at[buf], sem_q.at[buf])
    return cp, cq

  inflight = {k: start_gather(k) for k in range(NBUF)}
  lane = lax.iota(jnp.int32, LANES)

  for k in range(N_CHUNKS):
    cp, cq = inflight.pop(k)
    cp.wait()
    cq.wait()
    buf = k % NBUF

    def group(g, carry, k=k, buf=buf):
      rvec = g * LANES + lane

      def col(d, acc):
        cvec = (jnp.full((LANES,), d, jnp.int32) + lane) & (EMBED - 1)
        pv = plsc.load_gather(p_rows.at[buf], [rvec, cvec])
        qv = plsc.load_gather(q_rows.at[buf], [rvec, cvec])
        return acc + pv * qv

      acc = lax.fori_loop(0, EMBED, col, jnp.zeros((LANES,), jnp.float32),
                          unroll=8)
      out_v[pl.ds(k * CHUNK + g * LANES, LANES)] = acc
      return carry

    lax.fori_loop(0, CHUNK // LANES, group, None)

    if k + NBUF < N_CHUNKS:
      inflight[k + NBUF] = start_gather(k + NBUF)

  pltpu.sync_copy(out_v, y_hbm.at[pl.ds(base, B_PER_W)])


@jax.jit
def kernel(user_ids, item_ids, P, Q):
  mesh = plsc.VectorSubcoreMesh(core_axis_name="c", subcore_axis_name="s")
  run = pl.kernel(
      _body,
      out_type=jax.ShapeDtypeStruct((BATCH,), jnp.float32),
      mesh=mesh,
      scratch_types=[
          pltpu.VMEM((N_CHUNKS, CHUNK), jnp.int32),
          pltpu.VMEM((N_CHUNKS, CHUNK), jnp.int32),
          pltpu.VMEM((NBUF, CHUNK, EMBED), jnp.float32),
          pltpu.VMEM((NBUF, CHUNK, EMBED), jnp.float32),
          pltpu.VMEM((B_PER_W,), jnp.float32),
          pltpu.SemaphoreType.DMA,
          pltpu.SemaphoreType.DMA((NBUF,)),
          pltpu.SemaphoreType.DMA((NBUF,)),
      ],
      compiler_params=pltpu.CompilerParams(
          needs_layout_passes=False,
          skip_device_barrier=True,
          disable_bounds_checks=True,
      ),
  )
  return run(user_ids, item_ids, P, Q)


# speedup vs baseline: 1.0392x; 1.0392x over previous
import jax
import jax.numpy as jnp
from jax import lax
from jax.experimental import pallas as pl
from jax.experimental.pallas import tpu as pltpu
from jax.experimental.pallas import tpu_sc as plsc

BATCH = 16384
EMBED = 128
LANES = 16

NUM_CORES = 2
NUM_SUBCORES = 16
NUM_WORKERS = NUM_CORES * NUM_SUBCORES
B_PER_W = BATCH // NUM_WORKERS
CHUNK = 128
N_CHUNKS = B_PER_W // CHUNK
NBUF = 3


def _body(uid_hbm, iid_hbm, p_hbm, q_hbm, y_hbm,
          uidx_v, iidx_v, p_rows, q_rows, out_v, sem_idx, sem_p, sem_q):
  wid = lax.axis_index("s") * NUM_CORES + lax.axis_index("c")
  base = wid * B_PER_W

  idx_copies = []
  for k in range(N_CHUNKS):
    idx_copies.append(pltpu.async_copy(
        uid_hbm.at[pl.ds(base + k * CHUNK, CHUNK)], uidx_v.at[k], sem_idx))
    idx_copies.append(pltpu.async_copy(
        iid_hbm.at[pl.ds(base + k * CHUNK, CHUNK)], iidx_v.at[k], sem_idx))
  for c in idx_copies:
    c.wait()

  def start_gather(k):
    buf = k % NBUF
    cp = pltpu.async_copy(p_hbm.at[uidx_v.at[k]], p_rows.at[buf], sem_p.at[buf])
    cq = pltpu.async_copy(q_hbm.at[iidx_v.at[k]], q_rows.at[buf], sem_q.at[buf])
    return cp, cq

  inflight = {k: start_gather(k) for k in range(NBUF)}
  lane = lax.iota(jnp.int32, LANES)

  for k in range(N_CHUNKS):
    cp, cq = inflight.pop(k)
    cp.wait()
    cq.wait()
    buf = k % NBUF

    def group(g, carry, k=k, buf=buf):
      rvec = g * LANES + lane

      def col(d, acc):
        cvec = (jnp.full((LANES,), d, jnp.int32) + lane) & (EMBED - 1)
        pv = plsc.load_gather(p_rows.at[buf], [rvec, cvec])
        qv = plsc.load_gather(q_rows.at[buf], [rvec, cvec])
        return acc + pv * qv

      acc = lax.fori_loop(0, EMBED, col, jnp.zeros((LANES,), jnp.float32),
                          unroll=8)
      out_v[pl.ds(k * CHUNK + g * LANES, LANES)] = acc
      return carry

    lax.fori_loop(0, CHUNK // LANES, group, None)

    if k + NBUF < N_CHUNKS:
      inflight[k + NBUF] = start_gather(k + NBUF)

  pltpu.sync_copy(out_v, y_hbm.at[pl.ds(base, B_PER_W)])


@jax.jit
def kernel(user_ids, item_ids, P, Q):
  mesh = plsc.VectorSubcoreMesh(core_axis_name="c", subcore_axis_name="s")
  run = pl.kernel(
      _body,
      out_type=jax.ShapeDtypeStruct((BATCH,), jnp.float32),
      mesh=mesh,
      scratch_types=[
          pltpu.VMEM((N_CHUNKS, CHUNK), jnp.int32),
          pltpu.VMEM((N_CHUNKS, CHUNK), jnp.int32),
          pltpu.VMEM((NBUF, CHUNK, EMBED), jnp.float32),
          pltpu.VMEM((NBUF, CHUNK, EMBED), jnp.float32),
          pltpu.VMEM((B_PER_W,), jnp.float32),
          pltpu.SemaphoreType.DMA,
          pltpu.SemaphoreType.DMA((NBUF,)),
          pltpu.SemaphoreType.DMA((NBUF,)),
      ],
      compiler_params=pltpu.CompilerParams(
          needs_layout_passes=False,
          skip_device_barrier=True,
          disable_bounds_checks=True,
      ),
  )
  return run(user_ids, item_ids, P, Q)


# speedup vs baseline: 1.0674x; 1.0272x over previous
import jax
import jax.numpy as jnp
from jax import lax
from jax.experimental import pallas as pl
from jax.experimental.pallas import tpu as pltpu
from jax.experimental.pallas import tpu_sc as plsc

BATCH = 16384
EMBED = 128
LANES = 16

NUM_CORES = 2
NUM_SUBCORES = 16
NUM_WORKERS = NUM_CORES * NUM_SUBCORES
B_PER_W = BATCH // NUM_WORKERS
CHUNK = 64
N_CHUNKS = B_PER_W // CHUNK
NBUF = 4


def _body(uid_hbm, iid_hbm, p_hbm, q_hbm, y_hbm,
          uidx_v, iidx_v, p_rows, q_rows, out_v, sem_idx, sem_p, sem_q):
  wid = lax.axis_index("s") * NUM_CORES + lax.axis_index("c")
  base = wid * B_PER_W

  cu = pltpu.async_copy(uid_hbm.at[pl.ds(base, B_PER_W)], uidx_v, sem_idx)
  ci = pltpu.async_copy(iid_hbm.at[pl.ds(base, B_PER_W)], iidx_v, sem_idx)
  cu.wait()
  ci.wait()

  def start_gather(k):
    buf = k % NBUF
    cp = pltpu.async_copy(p_hbm.at[uidx_v.at[pl.ds(k * CHUNK, CHUNK)]],
                          p_rows.at[buf], sem_p.at[buf])
    cq = pltpu.async_copy(q_hbm.at[iidx_v.at[pl.ds(k * CHUNK, CHUNK)]],
                          q_rows.at[buf], sem_q.at[buf])
    return cp, cq

  inflight = {k: start_gather(k) for k in range(NBUF)}
  lane = lax.iota(jnp.int32, LANES)

  for k in range(N_CHUNKS):
    cp, cq = inflight.pop(k)
    cp.wait()
    cq.wait()
    buf = k % NBUF

    def group(g, carry, k=k, buf=buf):
      rvec = g * LANES + lane

      def col(d, acc):
        cvec = (jnp.full((LANES,), d, jnp.int32) + lane) & (EMBED - 1)
        pv = plsc.load_gather(p_rows.at[buf], [rvec, cvec])
        qv = plsc.load_gather(q_rows.at[buf], [rvec, cvec])
        return acc + pv * qv

      acc = lax.fori_loop(0, EMBED, col, jnp.zeros((LANES,), jnp.float32),
                          unroll=8)
      out_v[pl.ds(k * CHUNK + g * LANES, LANES)] = acc
      return carry

    lax.fori_loop(0, CHUNK // LANES, group, None)

    if k + NBUF < N_CHUNKS:
      inflight[k + NBUF] = start_gather(k + NBUF)

  pltpu.sync_copy(out_v, y_hbm.at[pl.ds(base, B_PER_W)])


@jax.jit
def kernel(user_ids, item_ids, P, Q):
  mesh = plsc.VectorSubcoreMesh(core_axis_name="c", subcore_axis_name="s")
  run = pl.kernel(
      _body,
      out_type=jax.ShapeDtypeStruct((BATCH,), jnp.float32),
      mesh=mesh,
      scratch_types=[
          pltpu.VMEM((B_PER_W,), jnp.int32),
          pltpu.VMEM((B_PER_W,), jnp.int32),
          pltpu.VMEM((NBUF, CHUNK, EMBED), jnp.float32),
          pltpu.VMEM((NBUF, CHUNK, EMBED), jnp.float32),
          pltpu.VMEM((B_PER_W,), jnp.float32),
          pltpu.SemaphoreType.DMA,
          pltpu.SemaphoreType.DMA((NBUF,)),
          pltpu.SemaphoreType.DMA((NBUF,)),
      ],
      compiler_params=pltpu.CompilerParams(
          needs_layout_passes=False,
          skip_device_barrier=True,
          disable_bounds_checks=True,
      ),
  )
  return run(user_ids, item_ids, P, Q)


# speedup vs baseline: 1.0778x; 1.0098x over previous
import jax
import jax.numpy as jnp
from jax import lax
from jax.experimental import pallas as pl
from jax.experimental.pallas import tpu as pltpu
from jax.experimental.pallas import tpu_sc as plsc

BATCH = 16384
EMBED = 128
LANES = 16

NUM_CORES = 2
NUM_SUBCORES = 16
NUM_WORKERS = NUM_CORES * NUM_SUBCORES
B_PER_W = BATCH // NUM_WORKERS
CHUNK = 64
N_CHUNKS = B_PER_W // CHUNK
NBUF = 4


def _body(uid_hbm, iid_hbm, p_hbm, q_hbm, y_hbm,
          uidx_v, iidx_v, p_rows, q_rows, out_v, sem_idx, sem_p, sem_q,
          sem_out):
  wid = lax.axis_index("s") * NUM_CORES + lax.axis_index("c")
  base = wid * B_PER_W

  cu0 = pltpu.async_copy(uid_hbm.at[pl.ds(base, CHUNK)],
                         uidx_v.at[pl.ds(0, CHUNK)], sem_idx)
  ci0 = pltpu.async_copy(iid_hbm.at[pl.ds(base, CHUNK)],
                         iidx_v.at[pl.ds(0, CHUNK)], sem_idx)
  REST = B_PER_W - CHUNK
  cur = pltpu.async_copy(uid_hbm.at[pl.ds(base + CHUNK, REST)],
                         uidx_v.at[pl.ds(CHUNK, REST)], sem_idx)
  cir = pltpu.async_copy(iid_hbm.at[pl.ds(base + CHUNK, REST)],
                         iidx_v.at[pl.ds(CHUNK, REST)], sem_idx)

  def start_gather(k):
    buf = k % NBUF
    cp = pltpu.async_copy(p_hbm.at[uidx_v.at[pl.ds(k * CHUNK, CHUNK)]],
                          p_rows.at[buf], sem_p.at[buf])
    cq = pltpu.async_copy(q_hbm.at[iidx_v.at[pl.ds(k * CHUNK, CHUNK)]],
                          q_rows.at[buf], sem_q.at[buf])
    return cp, cq

  cu0.wait()
  ci0.wait()
  inflight = {0: start_gather(0)}
  cur.wait()
  cir.wait()
  for k in range(1, NBUF):
    inflight[k] = start_gather(k)
  lane = lax.iota(jnp.int32, LANES)
  out_copies = []

  for k in range(N_CHUNKS):
    cp, cq = inflight.pop(k)
    cp.wait()
    cq.wait()
    buf = k % NBUF

    def group(g, carry, k=k, buf=buf):
      rvec = g * LANES + lane

      def col(d, acc):
        cvec = (jnp.full((LANES,), d, jnp.int32) + lane) & (EMBED - 1)
        pv = plsc.load_gather(p_rows.at[buf], [rvec, cvec])
        qv = plsc.load_gather(q_rows.at[buf], [rvec, cvec])
        return acc + pv * qv

      acc = lax.fori_loop(0, EMBED, col, jnp.zeros((LANES,), jnp.float32),
                          unroll=8)
      out_v[pl.ds(k * CHUNK + g * LANES, LANES)] = acc
      return carry

    lax.fori_loop(0, CHUNK // LANES, group, None)

    if k + NBUF < N_CHUNKS:
      inflight[k + NBUF] = start_gather(k + NBUF)

    out_copies.append(pltpu.async_copy(
        out_v.at[pl.ds(k * CHUNK, CHUNK)],
        y_hbm.at[pl.ds(base + k * CHUNK, CHUNK)], sem_out))

  for c in out_copies:
    c.wait()


@jax.jit
def kernel(user_ids, item_ids, P, Q):
  mesh = plsc.VectorSubcoreMesh(core_axis_name="c", subcore_axis_name="s")
  run = pl.kernel(
      _body,
      out_type=jax.ShapeDtypeStruct((BATCH,), jnp.float32),
      mesh=mesh,
      scratch_types=[
          pltpu.VMEM((B_PER_W,), jnp.int32),
          pltpu.VMEM((B_PER_W,), jnp.int32),
          pltpu.VMEM((NBUF, CHUNK, EMBED), jnp.float32),
          pltpu.VMEM((NBUF, CHUNK, EMBED), jnp.float32),
          pltpu.VMEM((B_PER_W,), jnp.float32),
          pltpu.SemaphoreType.DMA,
          pltpu.SemaphoreType.DMA((NBUF,)),
          pltpu.SemaphoreType.DMA((NBUF,)),
          pltpu.SemaphoreType.DMA,
      ],
      compiler_params=pltpu.CompilerParams(
          needs_layout_passes=False,
          skip_device_barrier=True,
          disable_bounds_checks=True,
      ),
  )
  return run(user_ids, item_ids, P, Q)
